# baseline (speedup 1.0000x reference)
import functools

import jax
import jax.numpy as jnp
from jax import lax
from jax.experimental import pallas as pl
from jax.experimental.pallas import tpu as pltpu
from jax.experimental.pallas import tpu_sc as plsc

N = 10000
D = 128
NP = 10240
NSC = 2
NTILE = 16
NW = NSC * NTILE
RPT = NP // NTILE
ZR = 320
BLK = 512
GRID = NP // BLK

def _mesh():
    return plsc.VectorSubcoreMesh(core_axis_name="c", subcore_axis_name="s")


SC0_FRAC = 0.39


def _chunks(E):
    ch0 = max(1, round(E * SC0_FRAC / (NTILE * 128)))
    ch1 = -(-(E - ch0 * NTILE * 128) // (NTILE * 128))
    return ch0, ch1


def _make_deg(CH0, CH1):
    CHM = max(CH0, CH1)

    @functools.partial(
        pl.kernel,
        mesh=_mesh(),
        out_type=jax.ShapeDtypeStruct((NSC, NP), jnp.float32),
        scratch_types=[
            pltpu.VMEM_SHARED((NP,), jnp.float32),
            pltpu.VMEM((CHM, 128), jnp.int32),
            pltpu.VMEM((128,), jnp.float32),
            pltpu.VMEM((RPT,), jnp.float32),
        ],
    )
    def deg_kernel(dst4, out, acc, dstv, onesv, zv):
        c = lax.axis_index("c")
        s = lax.axis_index("s")
        nch = jnp.where(c == 0, CH0, CH1)
        for k in range(8):
            onesv[pl.ds(k * 16, 16)] = jnp.ones((16,), jnp.float32)

        def zb(i, carry):
            zv[pl.ds(i * 16, 16)] = jnp.zeros((16,), jnp.float32)
            return carry

        lax.fori_loop(0, RPT // 16, zb, 0)
        pltpu.sync_copy(zv, acc.at[pl.ds(s * RPT, RPT)])
        pltpu.sync_copy(dst4.at[c].at[s], dstv)
        plsc.subcore_barrier()

        def body(j, carry):
            pltpu.sync_copy(onesv, acc.at[dstv.at[j]], add=True)
            return carry

        lax.fori_loop(0, nch, body, 0)
        plsc.subcore_barrier()
        pltpu.sync_copy(acc.at[pl.ds(s * RPT, RPT)], out.at[c].at[pl.ds(s * RPT, RPT)])

    return deg_kernel


def _make_prop(CH0, CH1):
    CHM = max(CH0, CH1)

    @functools.partial(
        pl.kernel,
        mesh=_mesh(),
        out_type=jax.ShapeDtypeStruct((NSC, NP, D), jnp.float32),
        scratch_types=[
            pltpu.VMEM_SHARED((NP, D), jnp.float32),
            pltpu.VMEM((CHM, 128), jnp.int32),
            pltpu.VMEM((CHM, 128), jnp.int32),
            pltpu.VMEM((128, D), jnp.float32),
            pltpu.SemaphoreType.DMA,
        ],
    )
    def prop_kernel(y, src4, dst4, out, acc, srcv, dstv, rows, sem):
        c = lax.axis_index("c")
        s = lax.axis_index("s")
        nch = jnp.where(c == 0, CH0, CH1)

        def zrow(i, carry):
            for k in range(D // 16):
                rows[i, pl.ds(k * 16, 16)] = jnp.zeros((16,), jnp.float32)
            return carry

        lax.fori_loop(0, 128, zrow, 0)
        base = s * RPT
        for t in range(RPT // 128):
            pltpu.sync_copy(rows, acc.at[pl.ds(base + t * 128, 128)])
        pltpu.sync_copy(src4.at[c].at[s], srcv)
        pltpu.sync_copy(dst4.at[c].at[s], dstv)
        plsc.subcore_barrier()

        def body(j, carry):
            pltpu.async_copy(y.at[srcv.at[j]], rows, sem).wait()
            pltpu.sync_copy(rows, acc.at[dstv.at[j]], add=True)
            return carry

        lax.fori_loop(0, nch, body, 0)
        plsc.subcore_barrier()
        pltpu.sync_copy(acc.at[pl.ds(base, RPT)], out.at[c].at[pl.ds(base, RPT)])

    return prop_kernel


def _mm1_body(xb, degb, w1, yout, dinvout):
    i = pl.program_id(0)
    t = jnp.dot(xb[...], w1[...], preferred_element_type=jnp.float32)
    degsum = degb[0, :] + degb[1, :] + 1.0
    rows = i * BLK + lax.broadcasted_iota(jnp.int32, (BLK,), 0)
    dinv = jnp.where(rows < N, lax.rsqrt(degsum), 0.0)
    yout[...] = t * dinv[:, None]
    dinvout[...] = dinv


def _mm1(xp, deg2, W1):
    return pl.pallas_call(
        _mm1_body,
        grid=(GRID,),
        in_specs=[
            pl.BlockSpec((BLK, D), lambda i: (i, 0)),
            pl.BlockSpec((NSC, BLK), lambda i: (0, i)),
            pl.BlockSpec((D, D), lambda i: (0, 0)),
        ],
        out_specs=[
            pl.BlockSpec((BLK, D), lambda i: (i, 0)),
            pl.BlockSpec((BLK,), lambda i: (i,)),
        ],
        out_shape=[
            jax.ShapeDtypeStruct((NP, D), jnp.float32),
            jax.ShapeDtypeStruct((NP,), jnp.float32),
        ],
    )(xp, deg2, W1)


def _mid_body(sb, y1b, dinvb, w2, b1, yout):
    dinv = dinvb[...]
    pre = (sb[0] + sb[1] + y1b[...]) * dinv[:, None] + b1[...]
    h = jnp.maximum(pre, 0.0)
    yout[...] = jnp.dot(h, w2[...], preferred_element_type=jnp.float32) * dinv[:, None]


def _mid(s1, y1p, dinvp, W2, b1):
    return pl.pallas_call(
        _mid_body,
        grid=(GRID,),
        in_specs=[
            pl.BlockSpec((NSC, BLK, D), lambda i: (0, i, 0)),
            pl.BlockSpec((BLK, D), lambda i: (i, 0)),
            pl.BlockSpec((BLK,), lambda i: (i,)),
            pl.BlockSpec((D, D), lambda i: (0, 0)),
            pl.BlockSpec((1, D), lambda i: (0, 0)),
        ],
        out_specs=pl.BlockSpec((BLK, D), lambda i: (i, 0)),
        out_shape=jax.ShapeDtypeStruct((NP, D), jnp.float32),
    )(s1, y1p, dinvp, W2, b1)


def _fin_body(sb, y2b, dinvb, b2, out):
    i = pl.program_id(0)
    v = (sb[0] + sb[1] + y2b[...]) * dinvb[...][:, None]
    part = jnp.sum(v, axis=0, keepdims=True) * (1.0 / N)

    @pl.when(i == 0)
    def _():
        out[...] = b2[...] + part

    @pl.when(i > 0)
    def _():
        out[...] = out[...] + part


def _fin(s2, y2p, dinvp, b2):
    return pl.pallas_call(
        _fin_body,
        grid=(GRID,),
        in_specs=[
            pl.BlockSpec((NSC, BLK, D), lambda i: (0, i, 0)),
            pl.BlockSpec((BLK, D), lambda i: (i, 0)),
            pl.BlockSpec((BLK,), lambda i: (i,)),
            pl.BlockSpec((1, D), lambda i: (0, 0)),
        ],
        out_specs=pl.BlockSpec((1, D), lambda i: (0, 0)),
        out_shape=jax.ShapeDtypeStruct((1, D), jnp.float32),
    )(s2, y2p, dinvp, b2)


def kernel(x, edge_index, W1, b1, W2, b2):
    E = edge_index.shape[1]
    CH0, CH1 = _chunks(E)
    CHM = max(CH0, CH1)
    L0 = NTILE * CH0 * 128
    L1 = NTILE * CH1 * 128
    EP = L0 + L1
    xp = jnp.zeros((NP, D), jnp.float32).at[:N].set(x)
    ei = edge_index
    if EP > E:
        ei = jnp.concatenate(
            [ei, jnp.full((2, EP - E), N, dtype=ei.dtype)], axis=1)
    e0 = ei[:, :L0].reshape(2, 1, NTILE, CH0, 128)
    e1 = ei[:, L0:].reshape(2, 1, NTILE, CH1, 128)
    e0 = jnp.pad(e0, ((0, 0), (0, 0), (0, 0), (0, CHM - CH0), (0, 0)),
                 constant_values=N)
    e1 = jnp.pad(e1, ((0, 0), (0, 0), (0, 0), (0, CHM - CH1), (0, 0)),
                 constant_values=N)
    e4 = jnp.concatenate([e0, e1], axis=1)
    src4, dst4 = e4[0], e4[1]

    deg2 = _make_deg(CH0, CH1)(dst4)
    y1p, dinvp = _mm1(xp, deg2, W1)
    prop = _make_prop(CH0, CH1)
    s1 = prop(y1p, src4, dst4)
    y2p = _mid(s1, y1p, dinvp, W2, b1.reshape(1, D))
    s2 = prop(y2p, src4, dst4)
    out = _fin(s2, y2p, dinvp, b2.reshape(1, D))
    return out.reshape(D)


# speedup vs baseline: 1.0790x; 1.0790x over previous
import functools

import jax
import jax.numpy as jnp
from jax import lax
from jax.experimental import pallas as pl
from jax.experimental.pallas import tpu as pltpu
from jax.experimental.pallas import tpu_sc as plsc

N = 10000
D = 128
NP = 10240
NSC = 2
NTILE = 16
NW = NSC * NTILE
RPT = NP // NTILE
ZR = 320
BLK = 512
GRID = NP // BLK

def _mesh():
    return plsc.VectorSubcoreMesh(core_axis_name="c", subcore_axis_name="s")


SC0_FRAC = 0.384


def _chunks(E):
    ch0 = max(1, round(E * SC0_FRAC / (NTILE * 128)))
    ch1 = -(-(E - ch0 * NTILE * 128) // (NTILE * 128))
    return ch0, ch1


def _make_deg(CH0, CH1):
    CHM = max(CH0, CH1)

    @functools.partial(
        pl.kernel,
        mesh=_mesh(),
        out_type=jax.ShapeDtypeStruct((NSC, NP), jnp.float32),
        scratch_types=[
            pltpu.VMEM_SHARED((NP,), jnp.float32),
            pltpu.VMEM((CHM, 128), jnp.int32),
            pltpu.VMEM((128,), jnp.float32),
            pltpu.VMEM((RPT,), jnp.float32),
        ],
    )
    def deg_kernel(dst4, out, acc, dstv, onesv, zv):
        c = lax.axis_index("c")
        s = lax.axis_index("s")
        nch = jnp.where(c == 0, CH0, CH1)
        for k in range(8):
            onesv[pl.ds(k * 16, 16)] = jnp.ones((16,), jnp.float32)

        def zb(i, carry):
            zv[pl.ds(i * 16, 16)] = jnp.zeros((16,), jnp.float32)
            return carry

        lax.fori_loop(0, RPT // 16, zb, 0)
        pltpu.sync_copy(zv, acc.at[pl.ds(s * RPT, RPT)])
        pltpu.sync_copy(dst4.at[c].at[s], dstv)
        plsc.subcore_barrier()

        def body(j, carry):
            pltpu.sync_copy(onesv, acc.at[dstv.at[j]], add=True)
            return carry

        lax.fori_loop(0, nch, body, 0)
        plsc.subcore_barrier()
        pltpu.sync_copy(acc.at[pl.ds(s * RPT, RPT)], out.at[c].at[pl.ds(s * RPT, RPT)])

    return deg_kernel


def _make_prop(CH0, CH1):
    CHM = max(CH0, CH1)

    @functools.partial(
        pl.kernel,
        mesh=_mesh(),
        out_type=jax.ShapeDtypeStruct((NSC, NP, D), jnp.float32),
        scratch_types=[
            pltpu.VMEM_SHARED((NP, D), jnp.float32),
            pltpu.VMEM((CHM, 128), jnp.int32),
            pltpu.VMEM((CHM, 128), jnp.int32),
            pltpu.VMEM((128, D), jnp.float32),
            pltpu.SemaphoreType.DMA,
        ],
    )
    def prop_kernel(y, src4, dst4, out, acc, srcv, dstv, rows, sem):
        c = lax.axis_index("c")
        s = lax.axis_index("s")
        nch = jnp.where(c == 0, CH0, CH1)

        def zrow(i, carry):
            for k in range(D // 16):
                rows[i, pl.ds(k * 16, 16)] = jnp.zeros((16,), jnp.float32)
            return carry

        lax.fori_loop(0, 128, zrow, 0)
        base = s * RPT
        for t in range(RPT // 128):
            pltpu.sync_copy(rows, acc.at[pl.ds(base + t * 128, 128)])
        pltpu.sync_copy(src4.at[c].at[s], srcv)
        pltpu.sync_copy(dst4.at[c].at[s], dstv)
        plsc.subcore_barrier()

        def body(j, carry):
            pltpu.async_copy(y.at[srcv.at[j]], rows, sem).wait()
            pltpu.sync_copy(rows, acc.at[dstv.at[j]], add=True)
            return carry

        lax.fori_loop(0, nch, body, 0)
        plsc.subcore_barrier()
        pltpu.sync_copy(acc.at[pl.ds(base, RPT)], out.at[c].at[pl.ds(base, RPT)])

    return prop_kernel


def _mm1_body(xb, degb, w1, yout, dinvout):
    i = pl.program_id(0)
    t = jnp.dot(xb[...], w1[...], preferred_element_type=jnp.float32)
    degsum = degb[0, :] + degb[1, :] + 1.0
    rows = i * BLK + lax.broadcasted_iota(jnp.int32, (BLK,), 0)
    dinv = jnp.where(rows < N, lax.rsqrt(degsum), 0.0)
    yout[...] = t * dinv[:, None]
    dinvout[...] = dinv


def _mm1(xp, deg2, W1):
    return pl.pallas_call(
        _mm1_body,
        grid=(GRID,),
        in_specs=[
            pl.BlockSpec((BLK, D), lambda i: (i, 0)),
            pl.BlockSpec((NSC, BLK), lambda i: (0, i)),
            pl.BlockSpec((D, D), lambda i: (0, 0)),
        ],
        out_specs=[
            pl.BlockSpec((BLK, D), lambda i: (i, 0)),
            pl.BlockSpec((BLK,), lambda i: (i,)),
        ],
        out_shape=[
            jax.ShapeDtypeStruct((NP, D), jnp.float32),
            jax.ShapeDtypeStruct((NP,), jnp.float32),
        ],
    )(xp, deg2, W1)


def _mid_body(sb, y1b, dinvb, w2, b1, yout):
    dinv = dinvb[...]
    pre = (sb[0] + sb[1] + y1b[...]) * dinv[:, None] + b1[...]
    h = jnp.maximum(pre, 0.0)
    yout[...] = jnp.dot(h, w2[...], preferred_element_type=jnp.float32) * dinv[:, None]


def _mid(s1, y1p, dinvp, W2, b1):
    return pl.pallas_call(
        _mid_body,
        grid=(GRID,),
        in_specs=[
            pl.BlockSpec((NSC, BLK, D), lambda i: (0, i, 0)),
            pl.BlockSpec((BLK, D), lambda i: (i, 0)),
            pl.BlockSpec((BLK,), lambda i: (i,)),
            pl.BlockSpec((D, D), lambda i: (0, 0)),
            pl.BlockSpec((1, D), lambda i: (0, 0)),
        ],
        out_specs=pl.BlockSpec((BLK, D), lambda i: (i, 0)),
        out_shape=jax.ShapeDtypeStruct((NP, D), jnp.float32),
    )(s1, y1p, dinvp, W2, b1)


def _fin_body(sb, y2b, dinvb, b2, out):
    i = pl.program_id(0)
    v = (sb[0] + sb[1] + y2b[...]) * dinvb[...][:, None]
    part = jnp.sum(v, axis=0, keepdims=True) * (1.0 / N)

    @pl.when(i == 0)
    def _():
        out[...] = b2[...] + part

    @pl.when(i > 0)
    def _():
        out[...] = out[...] + part


def _fin(s2, y2p, dinvp, b2):
    return pl.pallas_call(
        _fin_body,
        grid=(GRID,),
        in_specs=[
            pl.BlockSpec((NSC, BLK, D), lambda i: (0, i, 0)),
            pl.BlockSpec((BLK, D), lambda i: (i, 0)),
            pl.BlockSpec((BLK,), lambda i: (i,)),
            pl.BlockSpec((1, D), lambda i: (0, 0)),
        ],
        out_specs=pl.BlockSpec((1, D), lambda i: (0, 0)),
        out_shape=jax.ShapeDtypeStruct((1, D), jnp.float32),
    )(s2, y2p, dinvp, b2)


def kernel(x, edge_index, W1, b1, W2, b2):
    E = edge_index.shape[1]
    CH0, CH1 = _chunks(E)
    CHM = max(CH0, CH1)
    L0 = NTILE * CH0 * 128
    L1 = NTILE * CH1 * 128
    EP = L0 + L1
    xp = jnp.zeros((NP, D), jnp.float32).at[:N].set(x)
    ei = edge_index
    if EP > E:
        ei = jnp.concatenate(
            [ei, jnp.full((2, EP - E), N, dtype=ei.dtype)], axis=1)
    e0 = ei[:, :L0].reshape(2, 1, NTILE, CH0, 128)
    e1 = ei[:, L0:].reshape(2, 1, NTILE, CH1, 128)
    e0 = jnp.pad(e0, ((0, 0), (0, 0), (0, 0), (0, CHM - CH0), (0, 0)),
                 constant_values=N)
    e1 = jnp.pad(e1, ((0, 0), (0, 0), (0, 0), (0, CHM - CH1), (0, 0)),
                 constant_values=N)
    e4 = jnp.concatenate([e0, e1], axis=1)
    src4, dst4 = e4[0], e4[1]

    deg2 = _make_deg(CH0, CH1)(dst4)
    y1p, dinvp = _mm1(xp, deg2, W1)
    prop = _make_prop(CH0, CH1)
    s1 = prop(y1p, src4, dst4)
    y2p = _mid(s1, y1p, dinvp, W2, b1.reshape(1, D))
    s2 = prop(y2p, src4, dst4)
    out = _fin(s2, y2p, dinvp, b2.reshape(1, D))
    return out.reshape(D)
